# baseline (speedup 1.0000x reference)
import jax
import jax.numpy as jnp
from jax.experimental import pallas as pl
from jax.experimental.pallas import tpu as pltpu

_BM = 200
_NB = 5


def _gcn_kernel(x_ref, w_ref, b_ref, adj_hbm, out_ref, bufs, sems, support_ref):
    n = out_ref.shape[0]
    num = n // _BM
    n_outer = num // _NB

    def start_copy(idx, s):
        pltpu.make_async_copy(
            adj_hbm.at[pl.ds(idx * _BM, _BM), :],
            bufs.at[s],
            sems.at[s],
        ).start()

    for s in range(_NB):
        start_copy(s, s)

    support_ref[...] = jnp.dot(
        x_ref[...], w_ref[...], preferred_element_type=jnp.float32
    )

    def outer_body(o, carry):
        for s in range(_NB):
            idx = o * _NB + s
            pltpu.make_async_copy(
                adj_hbm.at[pl.ds(idx * _BM, _BM), :],
                bufs.at[s],
                sems.at[s],
            ).wait()
            out = jnp.dot(
                bufs[s], support_ref[...], preferred_element_type=jnp.float32
            )
            out = out + b_ref[...]
            m = jnp.max(out, axis=1, keepdims=True)
            shifted = out - m
            lse = jnp.log(jnp.sum(jnp.exp(shifted), axis=1, keepdims=True))
            out_ref[pl.ds(idx * _BM, _BM), :] = shifted - lse

            @pl.when(o + 1 < n_outer)
            def _():
                start_copy(idx + _NB, s)

        return carry

    jax.lax.fori_loop(0, n_outer, outer_body, 0)


def kernel(x, adj, W, b):
    n, nfeat = x.shape
    nclass = W.shape[1]
    b2 = b.reshape(1, nclass)

    return pl.pallas_call(
        _gcn_kernel,
        in_specs=[
            pl.BlockSpec(memory_space=pltpu.VMEM),
            pl.BlockSpec(memory_space=pltpu.VMEM),
            pl.BlockSpec(memory_space=pltpu.VMEM),
            pl.BlockSpec(memory_space=pl.ANY),
        ],
        out_specs=pl.BlockSpec(memory_space=pltpu.VMEM),
        out_shape=jax.ShapeDtypeStruct((n, nclass), jnp.float32),
        scratch_shapes=[
            pltpu.VMEM((_NB, _BM, n), jnp.float32),
            pltpu.SemaphoreType.DMA((_NB,)),
            pltpu.VMEM((n, nclass), jnp.float32),
        ],
        compiler_params=pltpu.CompilerParams(
            vmem_limit_bytes=100 * 1024 * 1024,
        ),
    )(x, W, b2, adj)


# speedup vs baseline: 1.0411x; 1.0411x over previous
import jax
import jax.numpy as jnp
from jax.experimental import pallas as pl
from jax.experimental.pallas import tpu as pltpu

_BM = 400


def _gcn_kernel(x_ref, w_ref, b_ref, adj_ref, out_ref):
    part = jnp.dot(adj_ref[...], x_ref[...], preferred_element_type=jnp.float32)
    out = jnp.dot(part, w_ref[...], preferred_element_type=jnp.float32)
    out = out + b_ref[...]
    m = jnp.max(out, axis=1, keepdims=True)
    shifted = out - m
    lse = jnp.log(jnp.sum(jnp.exp(shifted), axis=1, keepdims=True))
    out_ref[...] = shifted - lse


def kernel(x, adj, W, b):
    n, nfeat = x.shape
    nclass = W.shape[1]
    b2 = b.reshape(1, nclass)
    num_m = n // _BM

    return pl.pallas_call(
        _gcn_kernel,
        grid=(num_m,),
        in_specs=[
            pl.BlockSpec((n, nfeat), lambda i: (0, 0)),
            pl.BlockSpec((nfeat, nclass), lambda i: (0, 0)),
            pl.BlockSpec((1, nclass), lambda i: (0, 0)),
            pl.BlockSpec((_BM, n), lambda i: (i, 0)),
        ],
        out_specs=pl.BlockSpec((_BM, nclass), lambda i: (i, 0)),
        out_shape=jax.ShapeDtypeStruct((n, nclass), jnp.float32),
        compiler_params=pltpu.CompilerParams(
            dimension_semantics=("parallel",),
        ),
    )(x, W, b2, adj)
